# Initial kernel scaffold; baseline (speedup 1.0000x reference)
import jax
import jax.numpy as jnp
from jax.experimental import pallas as pl


def kernel(x, old_table, new_table):
    raise NotImplementedError("write your pallas kernel here")


# speedup vs baseline: 1.0371x; 1.0371x over previous
import functools

import jax
import jax.numpy as jnp
from jax import lax
from jax.experimental import pallas as pl
from jax.experimental.pallas import tpu as pltpu
from jax.experimental.pallas import tpu_sc as plsc


def _build_kernel(N, D, NW):
    n_w = N // NW
    IDXW = 128
    rows_idx = n_w // IDXW
    K = 5
    C = K * IDXW
    n_chunks = rows_idx // K

    mesh = plsc.VectorSubcoreMesh(core_axis_name="c", subcore_axis_name="s")

    @functools.partial(
        pl.kernel,
        mesh=mesh,
        out_type=jax.ShapeDtypeStruct((N, D), jnp.float32),
        compiler_params=pltpu.CompilerParams(use_tc_tiling_on_sc=False),
        scratch_types=[
            pltpu.VMEM((rows_idx, IDXW), jnp.int32),
            pltpu.VMEM((C, D), jnp.float32),
            pltpu.VMEM((C, D), jnp.float32),
            pltpu.SemaphoreType.DMA,
            pltpu.SemaphoreType.DMA,
        ],
    )
    def k(x_hbm, old_hbm, new_hbm, out_hbm, idx_v, rows_a, rows_b, sem_a, sem_b):
        wid = lax.axis_index("s") * 2 + lax.axis_index("c")
        base = wid * n_w
        pltpu.sync_copy(x_hbm.at[wid], idx_v)

        def chunk_body(c, carry):
            cps = []
            for kk in range(K):
                r = c * K + kk
                cps.append(pltpu.async_copy(
                    old_hbm.at[idx_v.at[r]],
                    rows_a.at[pl.ds(kk * IDXW, IDXW)], sem_a))
                cps.append(pltpu.async_copy(
                    new_hbm.at[idx_v.at[r]],
                    rows_b.at[pl.ds(kk * IDXW, IDXW)], sem_b))
            for cp in cps:
                cp.wait()

            def add_row(row, carry2):
                for col in range(0, D, 16):
                    rows_a[row, pl.ds(col, 16)] = (
                        rows_a[row, pl.ds(col, 16)] + rows_b[row, pl.ds(col, 16)])
                return carry2

            lax.fori_loop(0, C, add_row, 0)
            pltpu.sync_copy(rows_a, out_hbm.at[pl.ds(base + c * C, C)])
            return carry

        lax.fori_loop(0, n_chunks, chunk_body, 0)

    return k


def kernel(x, old_table, new_table):
    B, L = x.shape
    _, D = old_table.shape
    N = B * L
    NW = 32
    xf = x.reshape(-1).astype(jnp.int32).reshape(NW, N // NW // 128, 128)
    k = _build_kernel(N, D, NW)
    out = k(xf, old_table, new_table)
    return out.reshape(B, L, D)
